# baseline (speedup 1.0000x reference)
import functools

import jax
import jax.numpy as jnp
from jax import lax
from jax.experimental import pallas as pl
from jax.experimental.pallas import tpu as pltpu
from jax.experimental.pallas import tpu_sc as plsc

_B = 16384
_D = 128
_LANES = 16
_NC = 2
_NS = 16
_NW = _NC * _NS
_ROWS_PER_W = _B // _NW
_CHUNK = 128
_NCHUNK = _ROWS_PER_W // _CHUNK


def _sc_gather_mult(in_repr, idx2d, lambdas):
    mesh = plsc.VectorSubcoreMesh(core_axis_name="c", subcore_axis_name="s")

    @functools.partial(
        pl.kernel,
        out_type=jax.ShapeDtypeStruct((_B, _D), jnp.float32),
        mesh=mesh,
        scratch_types=[
            pltpu.VMEM((_NCHUNK, _CHUNK), jnp.int32),
            pltpu.VMEM((_CHUNK, _D), jnp.float32),
            pltpu.VMEM((_CHUNK, _D), jnp.float32),
            pltpu.VMEM((_CHUNK, _D), jnp.float32),
            pltpu.VMEM((_CHUNK, _D), jnp.float32),
            pltpu.SemaphoreType.DMA,
            pltpu.SemaphoreType.DMA,
            pltpu.SemaphoreType.DMA,
            pltpu.SemaphoreType.DMA,
            pltpu.SemaphoreType.DMA,
            pltpu.SemaphoreType.DMA,
        ],
    )
    def k(in_hbm, idx_hbm, lam_hbm, out_hbm, idx_v,
          lam0, x0, lam1, x1, g0, x0s, o0, g1, x1s, o1):
        wid = lax.axis_index("s") * _NC + lax.axis_index("c")
        base = wid * _ROWS_PER_W
        pltpu.sync_copy(idx_hbm.at[pl.ds(wid * _NCHUNK, _NCHUNK)], idx_v)

        bufs = [(lam0, x0, g0, x0s, o0), (lam1, x1, g1, x1s, o1)]
        gets = [None] * _NCHUNK
        puts = [None] * _NCHUNK

        def start(c):
            lam, xv, gs, xs, _ = bufs[c % 2]
            off = base + c * _CHUNK
            gets[c] = (
                pltpu.async_copy(lam_hbm.at[idx_v.at[c]], lam, gs),
                pltpu.async_copy(in_hbm.at[pl.ds(off, _CHUNK)], xv, xs),
            )

        start(0)
        for c in range(_NCHUNK):
            lam, xv, gs, xs, os = bufs[c % 2]
            if c + 1 < _NCHUNK:
                if c - 1 >= 0:
                    puts[c - 1].wait()
                start(c + 1)
            for cp in gets[c]:
                cp.wait()

            @pl.loop(0, _CHUNK, step=4)
            def _(r):
                for dr in range(4):
                    for c0 in range(0, _D, _LANES):
                        lam[r + dr, pl.ds(c0, _LANES)] = (
                            lam[r + dr, pl.ds(c0, _LANES)]
                            * xv[r + dr, pl.ds(c0, _LANES)]
                        )

            off = base + c * _CHUNK
            puts[c] = pltpu.async_copy(lam, out_hbm.at[pl.ds(off, _CHUNK)], os)

        puts[_NCHUNK - 2].wait()
        puts[_NCHUNK - 1].wait()

    return k(in_repr, idx2d, lambdas)


def kernel(in_repr, group_idx, lambdas):
    idx2d = group_idx.astype(jnp.int32).reshape(_B // _CHUNK, _CHUNK)
    return _sc_gather_mult(in_repr, idx2d, lambdas)


# speedup vs baseline: 1.0243x; 1.0243x over previous
import functools

import jax
import jax.numpy as jnp
from jax import lax
from jax.experimental import pallas as pl
from jax.experimental.pallas import tpu as pltpu
from jax.experimental.pallas import tpu_sc as plsc

_B = 16384
_D = 128
_LANES = 16
_NC = 2
_NS = 16
_NW = _NC * _NS
_ROWS_PER_W = _B // _NW
_CHUNK = 128
_NCHUNK = _ROWS_PER_W // _CHUNK


def _sc_gather_mult(in_repr, idx2d, lambdas):
    mesh = plsc.VectorSubcoreMesh(core_axis_name="c", subcore_axis_name="s")

    lam_scratch = [pltpu.VMEM((_CHUNK, _D), jnp.float32) for _ in range(_NCHUNK)]
    x_scratch = [pltpu.VMEM((_CHUNK, _D), jnp.float32) for _ in range(2)]
    sems = [pltpu.SemaphoreType.DMA for _ in range(2 * _NCHUNK + 2)]

    @functools.partial(
        pl.kernel,
        out_type=jax.ShapeDtypeStruct((_B, _D), jnp.float32),
        mesh=mesh,
        scratch_types=(
            [pltpu.VMEM((_NCHUNK, _CHUNK), jnp.int32)]
            + lam_scratch + x_scratch + sems
        ),
    )
    def k(in_hbm, idx_hbm, lam_hbm, out_hbm, idx_v, *bufs):
        lam = list(bufs[:_NCHUNK])
        xb = list(bufs[_NCHUNK:_NCHUNK + 2])
        gsem = list(bufs[_NCHUNK + 2:2 * _NCHUNK + 2])
        xsem = list(bufs[2 * _NCHUNK + 2:2 * _NCHUNK + 4])
        osem = list(bufs[2 * _NCHUNK + 4:])

        wid = lax.axis_index("s") * _NC + lax.axis_index("c")
        base = wid * _ROWS_PER_W
        pltpu.sync_copy(idx_hbm.at[pl.ds(wid * _NCHUNK, _NCHUNK)], idx_v)

        gets = [
            pltpu.async_copy(lam_hbm.at[idx_v.at[c]], lam[c], gsem[c])
            for c in range(_NCHUNK)
        ]
        xgets = [None] * _NCHUNK
        puts = [None] * _NCHUNK

        def start_x(c):
            xgets[c] = pltpu.async_copy(
                in_hbm.at[pl.ds(base + c * _CHUNK, _CHUNK)], xb[c % 2], xsem[c % 2]
            )

        start_x(0)
        for c in range(_NCHUNK):
            xv = xb[c % 2]
            if c + 1 < _NCHUNK:
                start_x(c + 1)
            gets[c].wait()
            xgets[c].wait()

            @pl.loop(0, _CHUNK)
            def _(r):
                for c0 in range(0, _D, _LANES):
                    lam[c][r, pl.ds(c0, _LANES)] = (
                        lam[c][r, pl.ds(c0, _LANES)] * xv[r, pl.ds(c0, _LANES)]
                    )

            puts[c] = pltpu.async_copy(
                lam[c], out_hbm.at[pl.ds(base + c * _CHUNK, _CHUNK)], osem[c % 2]
            )
        for c in range(_NCHUNK):
            puts[c].wait()

    return k(in_repr, idx2d, lambdas)


def kernel(in_repr, group_idx, lambdas):
    idx2d = group_idx.astype(jnp.int32).reshape(_B // _CHUNK, _CHUNK)
    return _sc_gather_mult(in_repr, idx2d, lambdas)


# speedup vs baseline: 1.0549x; 1.0299x over previous
import functools

import jax
import jax.numpy as jnp
from jax import lax
from jax.experimental import pallas as pl
from jax.experimental.pallas import tpu as pltpu
from jax.experimental.pallas import tpu_sc as plsc

_B = 16384
_D = 128
_LANES = 16
_NC = 2
_NS = 16
_NW = _NC * _NS
_ROWS_PER_W = _B // _NW
_CHUNK = 128
_NCHUNK = _ROWS_PER_W // _CHUNK


def _sc_gather_mult(in_repr, idx2d, lambdas):
    mesh = plsc.VectorSubcoreMesh(core_axis_name="c", subcore_axis_name="s")

    lam_scratch = [pltpu.VMEM((_CHUNK, _D), jnp.float32) for _ in range(_NCHUNK)]
    x_scratch = [pltpu.VMEM((_CHUNK, _D), jnp.float32) for _ in range(2)]
    sems = [pltpu.SemaphoreType.DMA for _ in range(2 * _NCHUNK + 2)]

    @functools.partial(
        pl.kernel,
        out_type=jax.ShapeDtypeStruct((_B, _D), jnp.float32),
        mesh=mesh,
        scratch_types=(
            [pltpu.VMEM((_NCHUNK, _CHUNK), jnp.int32)]
            + lam_scratch + x_scratch + sems
        ),
    )
    def k(in_hbm, idx_hbm, lam_hbm, out_hbm, idx_v, *bufs):
        lam = list(bufs[:_NCHUNK])
        xb = list(bufs[_NCHUNK:_NCHUNK + 2])
        gsem = list(bufs[_NCHUNK + 2:2 * _NCHUNK + 2])
        xsem = list(bufs[2 * _NCHUNK + 2:2 * _NCHUNK + 4])
        osem = list(bufs[2 * _NCHUNK + 4:])

        wid = lax.axis_index("s") * _NC + lax.axis_index("c")
        base = wid * _ROWS_PER_W

        xgets = [None] * _NCHUNK
        puts = [None] * _NCHUNK

        def start_x(c):
            xgets[c] = pltpu.async_copy(
                in_hbm.at[pl.ds(base + c * _CHUNK, _CHUNK)], xb[c % 2], xsem[c % 2]
            )

        start_x(0)
        pltpu.sync_copy(idx_hbm.at[pl.ds(wid * _NCHUNK, _NCHUNK)], idx_v)

        gets = [
            pltpu.async_copy(lam_hbm.at[idx_v.at[c]], lam[c], gsem[c])
            for c in range(_NCHUNK)
        ]
        for c in range(_NCHUNK):
            xv = xb[c % 2]
            if c + 1 < _NCHUNK:
                start_x(c + 1)
            gets[c].wait()
            xgets[c].wait()

            @pl.loop(0, _CHUNK)
            def _(r):
                for c0 in range(0, _D, _LANES):
                    lam[c][r, pl.ds(c0, _LANES)] = (
                        lam[c][r, pl.ds(c0, _LANES)] * xv[r, pl.ds(c0, _LANES)]
                    )

            puts[c] = pltpu.async_copy(
                lam[c], out_hbm.at[pl.ds(base + c * _CHUNK, _CHUNK)], osem[c % 2]
            )
        for c in range(_NCHUNK):
            puts[c].wait()

    return k(in_repr, idx2d, lambdas)


def kernel(in_repr, group_idx, lambdas):
    idx2d = group_idx.astype(jnp.int32).reshape(_B // _CHUNK, _CHUNK)
    return _sc_gather_mult(in_repr, idx2d, lambdas)
